# baseline (speedup 1.0000x reference)
import jax
import jax.numpy as jnp
from jax.experimental import pallas as pl
from jax.experimental.pallas import tpu as pltpu

_EPS = 1e-8
_B = 4096
_D = 256
_BLK = 512
_R = _B // _BLK
_CH = 512
_NCH = _B // _CH


def _koleo_body(x_ref, acc_ref, xn_ref):
    i = pl.program_id(0)

    @pl.when(i == 0)
    def _init():
        x = x_ref[...]
        nrm2 = jax.lax.dot_general(
            x * x, jnp.ones((_D, 1), jnp.float32), (((1,), (0,)), ((), ())),
            preferred_element_type=jnp.float32)
        inv = 1.0 / jnp.maximum(jnp.sqrt(nrm2), _EPS)
        xn_ref[...] = x * inv

    xi = xn_ref[pl.ds(i * _BLK, _BLK), :]
    diag = (jax.lax.broadcasted_iota(jnp.int32, (_BLK, _CH), 0)
            == jax.lax.broadcasted_iota(jnp.int32, (_BLK, _CH), 1))

    mlanes = jnp.full((_BLK, 128), jnp.float32(-3e38))
    for kk in range(_NCH):
        k = jax.lax.rem(i + kk, _NCH)
        xc = xn_ref[pl.ds(k * _CH, _CH), :]
        dch = jax.lax.dot_general(
            xi, xc, (((1,), (1,)), ((), ())),
            preferred_element_type=jnp.float32)
        if kk == 0:
            dch = jnp.where(diag, jnp.float32(-1.0), dch)
        r = dch.reshape(_BLK, _CH // 128, 128)
        mlanes = jnp.maximum(mlanes, jnp.max(r, axis=1))
    m = jnp.max(mlanes, axis=1, keepdims=True)

    si = jnp.sum(xi, axis=1, keepdims=True)
    qi = jnp.sum(xi * xi, axis=1, keepdims=True)
    dist2 = qi + (2.0 * _EPS) * si + _D * _EPS * _EPS + 1.0 - 2.0 * m
    dist = jnp.sqrt(jnp.maximum(dist2, 0.0))
    part = jnp.sum(jnp.log(dist + _EPS))

    @pl.when(i == 0)
    def _first():
        acc_ref[0, 0] = part

    @pl.when(i > 0)
    def _rest():
        acc_ref[0, 0] = acc_ref[0, 0] + part


def kernel(student_output):
    acc = pl.pallas_call(
        _koleo_body,
        grid=(_R,),
        in_specs=[pl.BlockSpec((_B, _D), lambda i: (0, 0))],
        out_specs=pl.BlockSpec(
            block_shape=(1, 1),
            index_map=lambda i: (0, 0),
            memory_space=pltpu.SMEM,
        ),
        out_shape=jax.ShapeDtypeStruct((1, 1), jnp.float32),
        scratch_shapes=[
            pltpu.VMEM((_B, _D), jnp.float32),
        ],
        compiler_params=pltpu.CompilerParams(
            dimension_semantics=("arbitrary",)),
    )(student_output)
    return -(acc[0, 0] / _B)


# speedup vs baseline: 3.4843x; 3.4843x over previous
import jax
import jax.numpy as jnp
from jax.experimental import pallas as pl
from jax.experimental.pallas import tpu as pltpu

_EPS = 1e-8
_B = 4096
_D = 256
_BLK = 512
_R = _B // _BLK
_CH = 512
_NCH = _B // _CH


def _koleo_body(x_ref, acc_ref, xn_ref):
    i = pl.program_id(0)

    @pl.when(i == 0)
    def _init():
        x = x_ref[...]
        nrm2 = jax.lax.dot_general(
            x * x, jnp.ones((_D, 1), jnp.float32), (((1,), (0,)), ((), ())),
            preferred_element_type=jnp.float32)
        inv = 1.0 / jnp.maximum(jnp.sqrt(nrm2), _EPS)
        xn_ref[...] = x * inv

    xi = xn_ref[pl.ds(i * _BLK, _BLK), :]
    diag = (jax.lax.broadcasted_iota(jnp.int32, (_BLK, _CH), 0)
            == jax.lax.broadcasted_iota(jnp.int32, (_BLK, _CH), 1))

    mlanes = jnp.full((_BLK, 128), jnp.float32(-3e38))
    for kk in range(_NCH):
        k = jax.lax.rem(i + kk, _NCH)
        xc = xn_ref[pl.ds(k * _CH, _CH), :]
        dch = jax.lax.dot_general(
            xi, xc, (((1,), (1,)), ((), ())),
            preferred_element_type=jnp.float32)
        if kk == 0:
            dch = jnp.where(diag, jnp.float32(-1.0), dch)
        m4 = jnp.maximum(
            jnp.maximum(dch[:, 0:128], dch[:, 128:256]),
            jnp.maximum(dch[:, 256:384], dch[:, 384:512]))
        mlanes = jnp.maximum(mlanes, m4)
    m = jnp.max(mlanes, axis=1, keepdims=True)

    si = jnp.sum(xi, axis=1, keepdims=True)
    qi = jnp.sum(xi * xi, axis=1, keepdims=True)
    dist2 = qi + (2.0 * _EPS) * si + _D * _EPS * _EPS + 1.0 - 2.0 * m
    dist = jnp.sqrt(jnp.maximum(dist2, 0.0))
    part = jnp.sum(jnp.log(dist + _EPS))

    @pl.when(i == 0)
    def _first():
        acc_ref[0, 0] = part

    @pl.when(i > 0)
    def _rest():
        acc_ref[0, 0] = acc_ref[0, 0] + part


def kernel(student_output):
    acc = pl.pallas_call(
        _koleo_body,
        grid=(_R,),
        in_specs=[pl.BlockSpec((_B, _D), lambda i: (0, 0))],
        out_specs=pl.BlockSpec(
            block_shape=(1, 1),
            index_map=lambda i: (0, 0),
            memory_space=pltpu.SMEM,
        ),
        out_shape=jax.ShapeDtypeStruct((1, 1), jnp.float32),
        scratch_shapes=[
            pltpu.VMEM((_B, _D), jnp.float32),
        ],
        compiler_params=pltpu.CompilerParams(
            dimension_semantics=("arbitrary",)),
    )(student_output)
    return -(acc[0, 0] / _B)
